# baseline (speedup 1.0000x reference)
import functools

import jax
import jax.numpy as jnp
from jax import lax
from jax.experimental import pallas as pl
from jax.experimental.pallas import tpu as pltpu
from jax.experimental.pallas import tpu_sc as plsc

K = 1_000_000
D = 16
L = 16
NC = 2
NS = 16
NW = NC * NS
B = 16384
BPW = B // NW
NV = BPW // L


def _sc_body(xt_hbm, a_hbm, mt_hbm, out_hbm, idx_v, xt_v, g_v, out_v, sem):
    wid = lax.axis_index("s") * NC + lax.axis_index("c")
    base = wid * BPW

    xcopies = [
        pltpu.async_copy(
            xt_hbm.at[pl.ds(d * B + base, BPW)],
            xt_v.at[pl.ds(d * BPW, BPW)],
            sem,
        )
        for d in range(D)
    ]
    pltpu.sync_copy(a_hbm.at[pl.ds(base, BPW)], idx_v.at[pl.ds(0, BPW)])

    for v in range(NV):
        av = idx_v[pl.ds(v * L, L)]
        for d in range(D - 1, -1, -1):
            idx_v[pl.ds(d * BPW + v * L, L)] = av + d * K

    GC = 128
    gcopies = [
        pltpu.async_copy(
            mt_hbm.at[idx_v.at[pl.ds(c * GC, GC)]],
            g_v.at[pl.ds(c * GC, GC)],
            sem,
        )
        for c in range(BPW * D // GC)
    ]
    for cp in xcopies:
        cp.wait()
    for cp in gcopies:
        cp.wait()

    for v in range(NV):
        acc = xt_v[pl.ds(v * L, L)] * g_v[pl.ds(v * L, L)]
        for d in range(1, D):
            o = d * BPW + v * L
            acc = acc + xt_v[pl.ds(o, L)] * g_v[pl.ds(o, L)]
        out_v[pl.ds(v * L, L)] = 1.0 / (1.0 + jnp.exp(-acc))

    pltpu.sync_copy(out_v, out_hbm.at[pl.ds(base, BPW)])


_sc_call = functools.partial(
    pl.kernel,
    out_type=jax.ShapeDtypeStruct((B,), jnp.float32),
    mesh=plsc.VectorSubcoreMesh(core_axis_name="c", subcore_axis_name="s"),
    scratch_types=[
        pltpu.VMEM((BPW * D,), jnp.int32),
        pltpu.VMEM((BPW * D,), jnp.float32),
        pltpu.VMEM((BPW * D,), jnp.float32),
        pltpu.VMEM((BPW,), jnp.float32),
        pltpu.SemaphoreType.DMA,
    ],
    compiler_params=pltpu.CompilerParams(
        needs_layout_passes=False, use_tc_tiling_on_sc=False
    ),
)(_sc_body)


@jax.jit
def kernel(X, A, m):
    xt_flat = X.T.reshape(-1)
    mt_flat = m.T.reshape(-1)
    return _sc_call(xt_flat, A.astype(jnp.int32), mt_flat)
